# Initial kernel scaffold; baseline (speedup 1.0000x reference)
import jax
import jax.numpy as jnp
from jax.experimental import pallas as pl


def kernel(x, w1, b1, w2, b2, w3, b3, w4, b4, w5, b5):
    raise NotImplementedError("write your pallas kernel here")


# speedup vs baseline: 8.1808x; 8.1808x over previous
import numpy as np
import jax
import jax.numpy as jnp
from jax.experimental import pallas as pl
from jax.experimental.pallas import tpu as pltpu

_B = 128


def _conv1_band_idx():
    IDX = np.zeros((5, 96, 256), np.int32)
    MSK = np.zeros((5, 96, 256), np.float32)
    for di in range(5):
        for w_in in range(32):
            for ci in range(3):
                k = w_in * 3 + ci
                for half in range(2):
                    for w2 in range(14):
                        dj = w_in - (2 * w2 + half)
                        if 0 <= dj < 5:
                            for co in range(8):
                                n = half * 128 + w2 * 8 + co
                                IDX[di, k, n] = ((di * 5 + dj) * 3 + ci) * 8 + co
                                MSK[di, k, n] = 1.0
    return IDX, MSK


def _conv2_band_idx():
    IDX = np.zeros((5, 128, 256), np.int32)
    MSK = np.zeros((5, 128, 256), np.float32)
    for di in range(5):
        for w_in in range(14):
            for ci in range(6):
                k = w_in * 8 + ci
                for half in range(2):
                    for w2 in range(5):
                        dj = w_in - (2 * w2 + half)
                        if 0 <= dj < 5:
                            for co in range(16):
                                n = half * 128 + w2 * 16 + co
                                IDX[di, k, n] = ((di * 5 + dj) * 6 + ci) * 16 + co
                                MSK[di, k, n] = 1.0
    return IDX, MSK


_IDX1, _MSK1 = _conv1_band_idx()
_IDX2, _MSK2 = _conv2_band_idx()
_BM1 = np.repeat((np.arange(32) % 16 < 14).astype(np.float32), 8)[None, :]
_BM2 = np.repeat((np.arange(16) % 8 < 5).astype(np.float32), 16)[None, :]


def _lenet_batch_kernel(x_ref, wb1_ref, b1_ref, wb2_ref, b2_ref,
                        w3_ref, b3_ref, w4_ref, b4_ref, w5_ref, b5_ref,
                        o_ref):
    B = x_ref.shape[0]

    acc = None
    for di in range(5):
        lhs = x_ref[:, di:di + 28, :].reshape(B * 28, 96)
        p = jnp.dot(lhs, wb1_ref[di], preferred_element_type=jnp.float32)
        acc = p if acc is None else acc + p
    y = acc + b1_ref[...]
    y = jnp.maximum(y[:, :128], y[:, 128:])
    y = jnp.maximum(y, 0.0)
    a1 = y.reshape(B, 14, 2, 128).max(axis=2)

    acc = None
    for di in range(5):
        lhs = a1[:, di:di + 10, :].reshape(B * 10, 128)
        p = jnp.dot(lhs, wb2_ref[di], preferred_element_type=jnp.float32)
        acc = p if acc is None else acc + p
    y = acc + b2_ref[...]
    y = jnp.maximum(y[:, :128], y[:, 128:])
    y = jnp.maximum(y, 0.0)
    a2 = y.reshape(B, 5, 2, 128).max(axis=2)

    acc = b3_ref[...]
    for h in range(5):
        acc = acc + jnp.dot(a2[:, h, :], w3_ref[h],
                            preferred_element_type=jnp.float32)
    h1 = jnp.maximum(acc, 0.0)
    h2 = jnp.maximum(jnp.dot(h1, w4_ref[...],
                             preferred_element_type=jnp.float32) + b4_ref[...],
                     0.0)
    o_ref[...] = jnp.dot(h2, w5_ref[...],
                         preferred_element_type=jnp.float32) + b5_ref[...]


def kernel(x, w1, b1, w2, b2, w3, b3, w4, b4, w5, b5):
    n = x.shape[0]
    xr = jnp.transpose(x.astype(jnp.float32), (0, 2, 3, 1)).reshape(n, 32, 96)

    wb1 = w1[:, :8].reshape(-1)[_IDX1] * _MSK1
    wb2 = w2[:, :16].reshape(-1)[_IDX2] * _MSK2
    w3p = jnp.pad(w3.reshape(5, 80, 128), ((0, 0), (0, 48), (0, 0)))
    b1L = jnp.tile(b1[:, :8], (1, 32)) * _BM1
    b2L = jnp.tile(b2[:, :16], (1, 16)) * _BM2

    grid = n // _B
    c2 = lambda i: (0, 0)
    c3 = lambda i: (0, 0, 0)
    out = pl.pallas_call(
        _lenet_batch_kernel,
        out_shape=jax.ShapeDtypeStruct((n, 128), jnp.float32),
        grid=(grid,),
        in_specs=[
            pl.BlockSpec((_B, 32, 96), lambda i: (i, 0, 0)),
            pl.BlockSpec((5, 96, 256), c3),
            pl.BlockSpec((1, 256), c2),
            pl.BlockSpec((5, 128, 256), c3),
            pl.BlockSpec((1, 256), c2),
            pl.BlockSpec((5, 128, 128), c3),
            pl.BlockSpec((1, 128), c2),
            pl.BlockSpec((128, 128), c2),
            pl.BlockSpec((1, 128), c2),
            pl.BlockSpec((128, 128), c2),
            pl.BlockSpec((1, 128), c2),
        ],
        out_specs=pl.BlockSpec((_B, 128), lambda i: (i, 0)),
        compiler_params=pltpu.CompilerParams(
            dimension_semantics=("parallel",),
            vmem_limit_bytes=64 * 1024 * 1024,
        ),
    )(xr, wb1, b1L, wb2, b2L, w3p, b3, w4, b4, w5, b5)

    return out[:, :100]


# speedup vs baseline: 130.8148x; 15.9905x over previous
import numpy as np
import jax
import jax.numpy as jnp
from jax.experimental import pallas as pl
from jax.experimental.pallas import tpu as pltpu

_B = 128


def _sel1():
    R = np.zeros((5, 2, 4, 2, 2), np.float32)
    for u in range(2):
        for r in range(4):
            for r2 in range(2):
                for hp in range(2):
                    di = 4 * u + r - 2 * r2 - hp
                    if 0 <= di < 5:
                        R[di, u, r, r2, hp] = 1.0
    C = np.zeros((5, 32, 2, 16), np.float32)
    for w_in in range(32):
        for wp in range(2):
            for w2 in range(14):
                dj = w_in - 2 * w2 - wp
                if 0 <= dj < 5:
                    C[dj, w_in, wp, w2] = 1.0
    return R, C


def _sel2():
    R = np.zeros((5, 3, 2, 2), np.float32)
    for t in range(3):
        for r2 in range(2):
            for hp in range(2):
                di = 2 * t + r2 - hp
                if 0 <= di < 5:
                    R[di, t, r2, hp] = 1.0
    C = np.zeros((5, 16, 2, 8), np.float32)
    for w in range(14):
        for wp in range(2):
            for w2 in range(5):
                dj = w - 2 * w2 - wp
                if 0 <= dj < 5:
                    C[dj, w, wp, w2] = 1.0
    return R, C


_R1, _C1 = _sel1()
_R2, _C2 = _sel2()
_BM1 = np.repeat((np.arange(128) % 16 < 14).astype(np.float32), 8)[None, :]
_BM2 = np.repeat((np.arange(32) % 8 < 5).astype(np.float32), 16)[None, :]


def _lenet_batch_kernel(x_ref, wq1_ref, b1_ref, wq2_ref, b2_ref,
                        w3_ref, b3_ref, w4_ref, b4_ref, w5_ref, b5_ref,
                        o_ref):
    B = x_ref.shape[2]

    pieces = [x_ref[ci, u:u + 7] for u in range(2) for ci in range(3)]
    lhs = jnp.concatenate(pieces, axis=2).reshape(7 * B, 768)
    y = jnp.dot(lhs, wq1_ref[...], preferred_element_type=jnp.float32)
    y = y + b1_ref[...]
    h0 = jnp.maximum(jnp.maximum(y[:, 0:128], y[:, 128:256]),
                     jnp.maximum(y[:, 256:384], y[:, 384:512]))
    h1 = jnp.maximum(jnp.maximum(y[:, 512:640], y[:, 640:768]),
                     jnp.maximum(y[:, 768:896], y[:, 896:1024]))
    a1 = jnp.maximum(jnp.concatenate([h0, h1], axis=1), 0.0)
    a1 = a1.reshape(7, B, 256)

    lhs2 = jnp.concatenate([a1[t:t + 5] for t in range(3)],
                           axis=2).reshape(5 * B, 768)
    y = jnp.dot(lhs2, wq2_ref[...], preferred_element_type=jnp.float32)
    y = y + b2_ref[...]
    a2 = jnp.maximum(jnp.maximum(jnp.maximum(y[:, 0:128], y[:, 128:256]),
                                 jnp.maximum(y[:, 256:384], y[:, 384:512])),
                     0.0)
    a2 = a2.reshape(5, B, 128)

    acc = b3_ref[...]
    for h in range(5):
        acc = acc + jnp.dot(a2[h], w3_ref[h],
                            preferred_element_type=jnp.float32)
    f1 = jnp.maximum(acc, 0.0)
    f2 = jnp.maximum(jnp.dot(f1, w4_ref[...],
                             preferred_element_type=jnp.float32) + b4_ref[...],
                     0.0)
    o_ref[...] = jnp.dot(f2, w5_ref[...],
                         preferred_element_type=jnp.float32) + b5_ref[...]


def kernel(x, w1, b1, w2, b2, w3, b3, w4, b4, w5, b5):
    n = x.shape[0]
    xg = x.astype(jnp.float32).reshape(n, 3, 8, 128).transpose(1, 2, 0, 3)

    w1t = w1[:, :8].reshape(5, 5, 3, 8)
    wq1 = jnp.einsum("djco,duryh,jwpv->ucrwyhpvo",
                     w1t, _R1, _C1).reshape(768, 1024)
    w2t = jnp.pad(w2[:, :16].reshape(5, 5, 6, 16),
                  ((0, 0), (0, 0), (0, 2), (0, 0)))
    wq2 = jnp.einsum("djco,dtyh,jwpv->tywchpvo",
                     w2t, _R2, _C2).reshape(768, 512)
    w3p = jnp.pad(w3.reshape(5, 80, 128), ((0, 0), (0, 48), (0, 0)))
    b1L = jnp.tile(b1[:, :8], (1, 128)) * _BM1
    b2L = jnp.tile(b2[:, :16], (1, 32)) * _BM2

    grid = n // _B
    c2 = lambda i: (0, 0)
    c3 = lambda i: (0, 0, 0)
    out = pl.pallas_call(
        _lenet_batch_kernel,
        out_shape=jax.ShapeDtypeStruct((n, 128), jnp.float32),
        grid=(grid,),
        in_specs=[
            pl.BlockSpec((3, 8, _B, 128), lambda i: (0, 0, i, 0)),
            pl.BlockSpec((768, 1024), c2),
            pl.BlockSpec((1, 1024), c2),
            pl.BlockSpec((768, 512), c2),
            pl.BlockSpec((1, 512), c2),
            pl.BlockSpec((5, 128, 128), c3),
            pl.BlockSpec((1, 128), c2),
            pl.BlockSpec((128, 128), c2),
            pl.BlockSpec((1, 128), c2),
            pl.BlockSpec((128, 128), c2),
            pl.BlockSpec((1, 128), c2),
        ],
        out_specs=pl.BlockSpec((_B, 128), lambda i: (i, 0)),
        compiler_params=pltpu.CompilerParams(
            dimension_semantics=("parallel",),
            vmem_limit_bytes=64 * 1024 * 1024,
        ),
    )(xg, wq1, b1L, wq2, b2L, w3p, b3, w4, b4, w5, b5)

    return out[:, :100]
